# baseline (speedup 1.0000x reference)
import functools
import jax
import jax.numpy as jnp
from jax import lax
from jax.experimental import pallas as pl
from jax.experimental.pallas import tpu as pltpu, tpu_sc as plsc

_NC = 2
_NS = 16
_NW = _NC * _NS
_LANES = 16
_CR = 8
_CC = 2048
_NBUF = 2


def _sc_add(nrows, ncols):
    rows_per_w = nrows // _NW
    col_chunks = ncols // _CC
    nchunks = (rows_per_w // _CR) * col_chunks
    mesh = plsc.VectorSubcoreMesh(core_axis_name="c", subcore_axis_name="s")

    @functools.partial(
        pl.kernel,
        out_type=jax.ShapeDtypeStruct((nrows, ncols), jnp.float32),
        mesh=mesh,
        scratch_types=[
            [pltpu.VMEM((_CR, _CC), jnp.float32) for _ in range(_NBUF)],
            [pltpu.VMEM((_CR, _CC), jnp.float32) for _ in range(_NBUF)],
            [pltpu.VMEM((_CR, _CC), jnp.float32) for _ in range(_NBUF)],
            [pltpu.VMEM_SHARED((_NS * _CR, _CC), jnp.float32) for _ in range(_NBUF)],
            [pltpu.SemaphoreType.DMA for _ in range(_NBUF)],
            [pltpu.SemaphoreType.DMA for _ in range(_NBUF)],
            [pltpu.SemaphoreType.DMA for _ in range(_NBUF)],
            [pltpu.SemaphoreType.DMA for _ in range(_NBUF)],
        ],
        compiler_params=pltpu.CompilerParams(use_tc_tiling_on_sc=True),
    )
    def k(x_hbm, p_hbm, o_hbm, bufx, bufp, bufo, spm, sx, sp, so1, so2):
        sid = lax.axis_index("s")
        wid = sid * _NC + lax.axis_index("c")
        row_base = wid * rows_per_w

        def slc(g):
            r0 = row_base + (g // col_chunks) * _CR
            c0 = (g % col_chunks) * _CC
            return (pl.ds(r0, _CR), pl.ds(c0, _CC))

        def load(g, b):
            s = slc(g)
            pltpu.async_copy(x_hbm.at[s], bufx[b], sx[b])
            pltpu.async_copy(p_hbm.at[s], bufp[b], sp[b])

        def wait_load(g, b):
            s = slc(g)
            pltpu.make_async_copy(x_hbm.at[s], bufx[b], sx[b]).wait()
            pltpu.make_async_copy(p_hbm.at[s], bufp[b], sp[b]).wait()

        def store1(b):
            pltpu.async_copy(bufo[b], spm[b].at[pl.ds(sid * _CR, _CR), :], so1[b])

        def wait_store1(b):
            pltpu.make_async_copy(bufo[b], spm[b].at[pl.ds(sid * _CR, _CR), :], so1[b]).wait()

        def store2(g, b):
            pltpu.async_copy(spm[b].at[pl.ds(sid * _CR, _CR), :], o_hbm.at[slc(g)], so2[b])

        def wait_store2(g, b):
            pltpu.make_async_copy(spm[b].at[pl.ds(sid * _CR, _CR), :], o_hbm.at[slc(g)], so2[b]).wait()

        for b in range(_NBUF):
            load(b, b)

        @pl.loop(0, nchunks // _NBUF)
        def trip(t):
            for b in range(_NBUF):
                g = t * _NBUF + b
                wait_load(g, b)

                def add_one(v):
                    s = pl.ds(v * _LANES, _LANES)
                    for r in range(_CR):
                        bufo[b][r, s] = bufx[b][r, s] + bufp[b][r, s]

                plsc.parallel_loop(0, _CC // _LANES, unroll=2)(add_one)

                @pl.when(g + _NBUF < nchunks)
                def _():
                    load(g + _NBUF, b)

                @pl.when(t > 0)
                def _():
                    wait_store2(g - _NBUF, b)

                store1(b)
                wait_store1(b)
                store2(g, b)

        for b in range(_NBUF):
            wait_store2(nchunks - _NBUF + b, b)

    return k


def kernel(x, pos_table):
    seq_len, d_model = x.shape
    return _sc_add(seq_len, d_model)(x, pos_table[:seq_len])


# speedup vs baseline: 1.0035x; 1.0035x over previous
import functools
import jax
import jax.numpy as jnp
from jax import lax
from jax.experimental import pallas as pl
from jax.experimental.pallas import tpu as pltpu, tpu_sc as plsc

_NC = 2
_NS = 16
_NW = _NC * _NS
_LANES = 16
_CR = 8
_CC = 2048
_NBUF = 2


def _sc_add(nrows, ncols):
    rows_per_w = nrows // _NW
    col_chunks = ncols // _CC
    nchunks = (rows_per_w // _CR) * col_chunks
    mesh = plsc.VectorSubcoreMesh(core_axis_name="c", subcore_axis_name="s")

    @functools.partial(
        pl.kernel,
        out_type=jax.ShapeDtypeStruct((nrows, ncols), jnp.float32),
        mesh=mesh,
        scratch_types=[
            [pltpu.VMEM((_CR, _CC), jnp.float32) for _ in range(_NBUF)],
            [pltpu.VMEM((_CR, _CC), jnp.float32) for _ in range(_NBUF)],
            [pltpu.VMEM((_CR, _CC), jnp.float32) for _ in range(_NBUF)],
            [pltpu.VMEM_SHARED((_NS * _CR, _CC), jnp.float32) for _ in range(_NBUF)],
            [pltpu.SemaphoreType.DMA for _ in range(_NBUF)],
            [pltpu.SemaphoreType.DMA for _ in range(_NBUF)],
            [pltpu.SemaphoreType.DMA for _ in range(_NBUF)],
            [pltpu.SemaphoreType.DMA for _ in range(_NBUF)],
        ],
        compiler_params=pltpu.CompilerParams(use_tc_tiling_on_sc=True),
    )
    def k(x_hbm, p_hbm, o_hbm, bufx, bufp, bufo, spm, sx, sp, so1, so2):
        sid = lax.axis_index("s")
        wid = sid * _NC + lax.axis_index("c")
        row_base = wid * rows_per_w

        def slc(g):
            r0 = row_base + (g // col_chunks) * _CR
            c0 = (g % col_chunks) * _CC
            return (pl.ds(r0, _CR), pl.ds(c0, _CC))

        def load(g, b):
            s = slc(g)
            pltpu.async_copy(x_hbm.at[s], bufx[b], sx[b])
            pltpu.async_copy(p_hbm.at[s], bufp[b], sp[b])

        def wait_load(g, b):
            s = slc(g)
            pltpu.make_async_copy(x_hbm.at[s], bufx[b], sx[b]).wait()
            pltpu.make_async_copy(p_hbm.at[s], bufp[b], sp[b]).wait()

        def store1(b):
            pltpu.async_copy(bufo[b], spm[b].at[pl.ds(sid * _CR, _CR), :], so1[b])

        def wait_store1(b):
            pltpu.make_async_copy(bufo[b], spm[b].at[pl.ds(sid * _CR, _CR), :], so1[b]).wait()

        def store2(g, b):
            pltpu.async_copy(spm[b].at[pl.ds(sid * _CR, _CR), :], o_hbm.at[slc(g)], so2[b])

        def wait_store2(g, b):
            pltpu.make_async_copy(spm[b].at[pl.ds(sid * _CR, _CR), :], o_hbm.at[slc(g)], so2[b]).wait()

        for b in range(_NBUF):
            load(b, b)

        @pl.loop(0, nchunks // _NBUF)
        def trip(t):
            for b in range(_NBUF):
                g = t * _NBUF + b
                ob = 1 - b
                wait_load(g, b)

                def drain_other():
                    wait_store1(ob)
                    store2(g - 1, ob)
                if b == 0:
                    pl.when(t > 0)(drain_other)
                else:
                    drain_other()

                def add_one(v):
                    s = pl.ds(v * _LANES, _LANES)
                    for r in range(_CR):
                        bufo[b][r, s] = bufx[b][r, s] + bufp[b][r, s]

                plsc.parallel_loop(0, _CC // _LANES, unroll=2)(add_one)

                @pl.when(g + _NBUF < nchunks)
                def _():
                    load(g + _NBUF, b)

                @pl.when(t > 0)
                def _():
                    wait_store2(g - _NBUF, b)

                store1(b)

        wait_store1(_NBUF - 1)
        store2(nchunks - 1, _NBUF - 1)
        for b in range(_NBUF):
            wait_store2(nchunks - _NBUF + b, b)

    return k


def kernel(x, pos_table):
    seq_len, d_model = x.shape
    return _sc_add(seq_len, d_model)(x, pos_table[:seq_len])
